# Initial kernel scaffold; baseline (speedup 1.0000x reference)
import jax
import jax.numpy as jnp
from jax.experimental import pallas as pl


def kernel(x, Wq, bq, Wk, bk, Wv, bv, Wo, bo, ln1_g, ln1_b, w1, b1, w2, b2, ln2_g, ln2_b, cw, cb, bn_g, bn_b):
    raise NotImplementedError("write your pallas kernel here")


# speedup vs baseline: 6.1405x; 6.1405x over previous
import math

import jax
import jax.numpy as jnp
from jax.experimental import pallas as pl

B, L, D = 2, 2048, 768
H, DK = 12, 64
HID = 2048
U = 32
NEG = float("-inf")
f32 = jnp.float32


def _dot(a, b):
    return jax.lax.dot_general(a, b, (((1,), (0,)), ((), ())),
                               preferred_element_type=f32)


def _dot_t(a, b):
    return jax.lax.dot_general(a, b, (((1,), (1,)), ((), ())),
                               preferred_element_type=f32)


def _dot_00(a, b):
    return jax.lax.dot_general(a, b, (((0,), (0,)), ((), ())),
                               preferred_element_type=f32)


def _elu(t):
    return jnp.where(t > 0, t, jnp.exp(t) - 1.0)


def _ln(t, g, b):
    mu = jnp.mean(t, axis=1, keepdims=True)
    var = jnp.mean((t - mu) ** 2, axis=1, keepdims=True)
    return (t - mu) * jax.lax.rsqrt(var + 1e-3) * g + b


def _qkv_body(x_ref, w_ref, b_ref, q_ref, k_ref, v_ref):
    r = _dot(x_ref[0], w_ref[...]) + b_ref[...]
    for h in range(H):
        q_ref[0, h] = r[:, h * DK:(h + 1) * DK]
        k_ref[0, h] = r[:, D + h * DK:D + (h + 1) * DK]
        v_ref[0, h] = r[:, 2 * D + h * DK:2 * D + (h + 1) * DK]


def _m_body(q_ref, k_ref, at_ref, m_ref):
    at = at_ref[...]
    msk = at > 0
    for b in range(B):
        for h in range(H):
            qh = q_ref[b, h]
            kh = k_ref[b, h]
            st = _dot_t(kh, qh)
            mx = jnp.max(jnp.where(msk, st, NEG), axis=0, keepdims=True)
            sm = jnp.sum(st * at, axis=0, keepdims=True) * (1.0 / U)
            m_ref[b * H + h:b * H + h + 1, :] = mx - sm


def _topk_body(m_ref, idx_ref):
    m = m_ref[...]
    lane_l = jax.lax.broadcasted_iota(jnp.int32, (1, L), 1)
    lane_u = jax.lax.broadcasted_iota(jnp.int32, (B * H, U), 1)

    def body(i, carry):
        m, idxs = carry
        mx = jnp.max(m, axis=1, keepdims=True)
        iv = jnp.min(jnp.where(m == mx, lane_l, L), axis=1, keepdims=True)
        idxs = jnp.where(lane_u == i, iv, idxs)
        m = jnp.where(lane_l == iv, NEG, m)
        return m, idxs

    _, idxs = jax.lax.fori_loop(
        0, U, body, (m, jnp.zeros((B * H, U), jnp.int32)))
    idx_ref[...] = idxs


def _attn_body(idx_ref, q_ref, k_ref, v_ref, wo_ref, bo_ref, o_ref):
    h = pl.program_id(1)
    q = q_ref[0, 0]
    k = k_ref[0, 0]
    v = v_ref[0, 0]
    col = jax.lax.broadcasted_iota(jnp.int32, (L, 1), 0)
    ot = (col == idx_ref[0]).astype(f32)
    qr = _dot_00(ot, q)
    s = _dot_t(qr, k) * (1.0 / math.sqrt(DK))
    s = s - jnp.max(s, axis=1, keepdims=True)
    e = jnp.exp(s)
    attn = e / jnp.sum(e, axis=1, keepdims=True)
    upd = _dot(attn, v)
    mv = jnp.mean(v, axis=0, keepdims=True)
    wo = wo_ref[0]
    contrib = _dot(ot, _dot(upd - mv, wo)) + _dot(mv, wo)

    @pl.when(h == 0)
    def _():
        o_ref[0] = contrib + bo_ref[...]

    @pl.when(h != 0)
    def _():
        o_ref[0] = o_ref[0] + contrib


def _ffn_body(x_ref, a_ref, w1_ref, b1_ref, w2_ref, b2_ref,
              g1_ref, bl1_ref, g2_ref, bl2_ref, o_ref):
    o1 = _ln(x_ref[0] + a_ref[0], g1_ref[...], bl1_ref[...])
    f = _elu(_dot(o1, w1_ref[...]) + b1_ref[...])
    f2 = _dot(f, w2_ref[...]) + b2_ref[...]
    o_ref[0] = _ln(o1 + f2, g2_ref[...], bl2_ref[...])


def _conv_body(y_ref, w0_ref, w1_ref, w2_ref, cb_ref, z_ref, s_ref, q_ref):
    t = y_ref[0]
    a = _dot(t, w0_ref[...])
    c = _dot(t, w2_ref[...])
    zero = jnp.zeros((1, D), f32)
    z = _dot(t, w1_ref[...]) + cb_ref[...]
    z = z + jnp.concatenate([zero, a[:-1, :]], axis=0)
    z = z + jnp.concatenate([c[1:, :], zero], axis=0)
    z_ref[0] = z
    s_ref[0] = jnp.sum(z, axis=0, keepdims=True)
    q_ref[0] = jnp.sum(z * z, axis=0, keepdims=True)


_FBLK = 128
_NFB = L // _FBLK

def _pool_body(zc_ref, zn_ref, s_ref, q_ref, g_ref, b_ref, o_ref):
    i = pl.program_id(1)
    n = float(B * L)
    mu = (s_ref[0] + s_ref[1]) * (1.0 / n)
    var = (q_ref[0] + q_ref[1]) * (1.0 / n) - mu * mu
    sc = g_ref[...] * jax.lax.rsqrt(var + 1e-3)
    zn = _elu((zc_ref[0] - mu) * sc + b_ref[...])
    nx = _elu((zn_ref[0, 0:1, :] - mu) * sc + b_ref[...])
    nx = jnp.where(i == _NFB - 1, NEG, nx)
    z3 = zn.reshape(_FBLK // 2, 2, D)
    ev = z3[:, 0, :]
    od = z3[:, 1, :]
    ev_n = jnp.concatenate([ev[1:, :], nx], axis=0)
    o_ref[0] = jnp.maximum(jnp.maximum(ev, od), ev_n)


def kernel(x, Wq, bq, Wk, bk, Wv, bv, Wo, bo, ln1_g, ln1_b, w1, b1, w2, b2,
           ln2_g, ln2_b, cw, cb, bn_g, bn_b):
    idx = jax.random.randint(jax.random.key(42), (L, U), 0, L)
    At = jnp.zeros((L, L), f32).at[idx, jnp.arange(L)[:, None]].add(1.0)

    Wqkv = jnp.concatenate(
        [Wq.reshape(D, H * DK), Wk.reshape(D, H * DK), Wv.reshape(D, H * DK)],
        axis=1)
    bqkv = jnp.concatenate(
        [bq.reshape(-1), bk.reshape(-1), bv.reshape(-1)])[None, :]

    hspec = pl.BlockSpec((1, H, 256, DK), lambda b, i: (b, 0, i, 0))
    hshape = jax.ShapeDtypeStruct((B, H, L, DK), f32)
    q, k, v = pl.pallas_call(
        _qkv_body,
        grid=(B, L // 256),
        in_specs=[
            pl.BlockSpec((1, 256, D), lambda b, i: (b, i, 0)),
            pl.BlockSpec((D, 3 * D), lambda b, i: (0, 0)),
            pl.BlockSpec((1, 3 * D), lambda b, i: (0, 0)),
        ],
        out_specs=[hspec, hspec, hspec],
        out_shape=[hshape, hshape, hshape],
    )(x, Wqkv, bqkv)

    M = pl.pallas_call(
        _m_body,
        grid=(L // 128,),
        in_specs=[
            pl.BlockSpec((B, H, 128, DK), lambda i: (0, 0, i, 0)),
            pl.BlockSpec((B, H, L, DK), lambda i: (0, 0, 0, 0)),
            pl.BlockSpec((L, 128), lambda i: (0, i)),
        ],
        out_specs=pl.BlockSpec((B * H, 128), lambda i: (0, i)),
        out_shape=jax.ShapeDtypeStruct((B * H, L), f32),
    )(q, k, At)

    top_idx = pl.pallas_call(
        _topk_body,
        in_specs=[pl.BlockSpec((B * H, L), lambda: (0, 0))],
        out_specs=pl.BlockSpec((B * H, U), lambda: (0, 0)),
        out_shape=jax.ShapeDtypeStruct((B * H, U), jnp.int32),
    )(M)

    attn_out = pl.pallas_call(
        _attn_body,
        grid=(B, H),
        in_specs=[
            pl.BlockSpec((1, 1, U), lambda b, h: (b * H + h, 0, 0)),
            pl.BlockSpec((1, 1, L, DK), lambda b, h: (b, h, 0, 0)),
            pl.BlockSpec((1, 1, L, DK), lambda b, h: (b, h, 0, 0)),
            pl.BlockSpec((1, 1, L, DK), lambda b, h: (b, h, 0, 0)),
            pl.BlockSpec((1, DK, D), lambda b, h: (h, 0, 0)),
            pl.BlockSpec((1, D), lambda b, h: (0, 0)),
        ],
        out_specs=pl.BlockSpec((1, L, D), lambda b, h: (b, 0, 0)),
        out_shape=jax.ShapeDtypeStruct((B, L, D), f32),
    )(top_idx.reshape(B * H, 1, U), q, k, v, Wo, bo[None, :])

    y2 = pl.pallas_call(
        _ffn_body,
        grid=(B, L // 256),
        in_specs=[
            pl.BlockSpec((1, 256, D), lambda b, i: (b, i, 0)),
            pl.BlockSpec((1, 256, D), lambda b, i: (b, i, 0)),
            pl.BlockSpec((D, HID), lambda b, i: (0, 0)),
            pl.BlockSpec((1, HID), lambda b, i: (0, 0)),
            pl.BlockSpec((HID, D), lambda b, i: (0, 0)),
            pl.BlockSpec((1, D), lambda b, i: (0, 0)),
            pl.BlockSpec((1, D), lambda b, i: (0, 0)),
            pl.BlockSpec((1, D), lambda b, i: (0, 0)),
            pl.BlockSpec((1, D), lambda b, i: (0, 0)),
            pl.BlockSpec((1, D), lambda b, i: (0, 0)),
        ],
        out_specs=pl.BlockSpec((1, 256, D), lambda b, i: (b, i, 0)),
        out_shape=jax.ShapeDtypeStruct((B, L, D), f32),
    )(x, attn_out, w1, b1[None, :], w2, b2[None, :],
      ln1_g[None, :], ln1_b[None, :], ln2_g[None, :], ln2_b[None, :])

    z, zsum, zsq = pl.pallas_call(
        _conv_body,
        grid=(B,),
        in_specs=[
            pl.BlockSpec((1, L, D), lambda b: (b, 0, 0)),
            pl.BlockSpec((D, D), lambda b: (0, 0)),
            pl.BlockSpec((D, D), lambda b: (0, 0)),
            pl.BlockSpec((D, D), lambda b: (0, 0)),
            pl.BlockSpec((1, D), lambda b: (0, 0)),
        ],
        out_specs=[
            pl.BlockSpec((1, L, D), lambda b: (b, 0, 0)),
            pl.BlockSpec((1, 1, D), lambda b: (b, 0, 0)),
            pl.BlockSpec((1, 1, D), lambda b: (b, 0, 0)),
        ],
        out_shape=[
            jax.ShapeDtypeStruct((B, L, D), f32),
            jax.ShapeDtypeStruct((B, 1, D), f32),
            jax.ShapeDtypeStruct((B, 1, D), f32),
        ],
    )(y2, cw[0], cw[1], cw[2], cb[None, :])

    out = pl.pallas_call(
        _pool_body,
        grid=(B, _NFB),
        in_specs=[
            pl.BlockSpec((1, _FBLK, D), lambda b, i: (b, i, 0)),
            pl.BlockSpec((1, _FBLK, D),
                         lambda b, i: (b, jnp.minimum(i + 1, _NFB - 1), 0)),
            pl.BlockSpec((B, 1, D), lambda b, i: (0, 0, 0)),
            pl.BlockSpec((B, 1, D), lambda b, i: (0, 0, 0)),
            pl.BlockSpec((1, D), lambda b, i: (0, 0)),
            pl.BlockSpec((1, D), lambda b, i: (0, 0)),
        ],
        out_specs=pl.BlockSpec((1, _FBLK // 2, D), lambda b, i: (b, i, 0)),
        out_shape=jax.ShapeDtypeStruct((B, L // 2, D), f32),
    )(z, z, zsum, zsq, bn_g[None, :], bn_b[None, :])

    return out


# speedup vs baseline: 9.3303x; 1.5195x over previous
import math

import jax
import jax.numpy as jnp
from jax.experimental import pallas as pl

B, L, D = 2, 2048, 768
H, DK = 12, 64
HID = 2048
U = 32
NEG = float("-inf")
f32 = jnp.float32


def _dot(a, b):
    return jax.lax.dot_general(a, b, (((1,), (0,)), ((), ())),
                               preferred_element_type=f32)


def _dot_t(a, b):
    return jax.lax.dot_general(a, b, (((1,), (1,)), ((), ())),
                               preferred_element_type=f32)


def _dot_00(a, b):
    return jax.lax.dot_general(a, b, (((0,), (0,)), ((), ())),
                               preferred_element_type=f32)


def _elu(t):
    return jnp.where(t > 0, t, jnp.exp(t) - 1.0)


def _ln(t, g, b):
    mu = jnp.mean(t, axis=1, keepdims=True)
    var = jnp.mean((t - mu) ** 2, axis=1, keepdims=True)
    return (t - mu) * jax.lax.rsqrt(var + 1e-3) * g + b


def _qkv_body(x_ref, w_ref, b_ref, q_ref, k_ref, v_ref):
    r = _dot(x_ref[0], w_ref[...]) + b_ref[...]
    for h in range(H):
        q_ref[0, h] = r[:, h * DK:(h + 1) * DK]
        k_ref[0, h] = r[:, D + h * DK:D + (h + 1) * DK]
        v_ref[0, h] = r[:, 2 * D + h * DK:2 * D + (h + 1) * DK]


def _m_body(q_ref, k_ref, at_ref, m_ref):
    at = at_ref[...]
    msk = at > 0
    for b in range(B):
        for h in range(H):
            qh = q_ref[b, h]
            kh = k_ref[b, h]
            st = _dot_t(kh, qh)
            mx = jnp.max(jnp.where(msk, st, NEG), axis=0, keepdims=True)
            sm = jnp.sum(st * at, axis=0, keepdims=True) * (1.0 / U)
            m_ref[b * H + h:b * H + h + 1, :] = mx - sm


def _topk_body(m_ref, idx_ref):
    m = m_ref[...]
    lane_l = jax.lax.broadcasted_iota(jnp.int32, (1, L), 1)
    lane_u = jax.lax.broadcasted_iota(jnp.int32, (B * H, U), 1)

    def body(i, carry):
        m, idxs = carry
        mx = jnp.max(m, axis=1, keepdims=True)
        iv = jnp.min(jnp.where(m == mx, lane_l, L), axis=1, keepdims=True)
        idxs = jnp.where(lane_u == i, iv, idxs)
        m = jnp.where(lane_l == iv, NEG, m)
        return m, idxs

    _, idxs = jax.lax.fori_loop(
        0, U, body, (m, jnp.zeros((B * H, U), jnp.int32)))
    idx_ref[...] = idxs


def _attn_body(idx_ref, q_ref, k_ref, v_ref, wo_ref, bo_ref, o_ref):
    h = pl.program_id(1)
    q = q_ref[0, 0]
    k = k_ref[0, 0]
    v = v_ref[0, 0]
    col = jax.lax.broadcasted_iota(jnp.int32, (L, 1), 0)
    ot = (col == idx_ref[0]).astype(f32)
    qr = _dot_00(ot, q)
    s = _dot_t(qr, k) * (1.0 / math.sqrt(DK))
    s = s - jnp.max(s, axis=1, keepdims=True)
    e = jnp.exp(s)
    attn = e / jnp.sum(e, axis=1, keepdims=True)
    upd = _dot(attn, v)
    mv = jnp.mean(v, axis=0, keepdims=True)
    wo = wo_ref[0]
    contrib = _dot(ot, _dot(upd - mv, wo)) + _dot(mv, wo)

    @pl.when(h == 0)
    def _():
        o_ref[0] = contrib + bo_ref[...]

    @pl.when(h != 0)
    def _():
        o_ref[0] = o_ref[0] + contrib


def _ffn_body(x_ref, a_ref, w1_ref, b1_ref, w2_ref, b2_ref,
              g1_ref, bl1_ref, g2_ref, bl2_ref, o_ref):
    o1 = _ln(x_ref[0] + a_ref[0], g1_ref[...], bl1_ref[...])
    f = _elu(_dot(o1, w1_ref[...]) + b1_ref[...])
    f2 = _dot(f, w2_ref[...]) + b2_ref[...]
    o_ref[0] = _ln(o1 + f2, g2_ref[...], bl2_ref[...])


def _conv_body(y_ref, w0_ref, w1_ref, w2_ref, cb_ref, z_ref, s_ref, q_ref):
    t = y_ref[0]
    a = _dot(t, w0_ref[...])
    c = _dot(t, w2_ref[...])
    zero = jnp.zeros((1, D), f32)
    z = _dot(t, w1_ref[...]) + cb_ref[...]
    z = z + jnp.concatenate([zero, a[:-1, :]], axis=0)
    z = z + jnp.concatenate([c[1:, :], zero], axis=0)
    z_ref[0] = z
    s_ref[0] = jnp.sum(z, axis=0, keepdims=True)
    q_ref[0] = jnp.sum(z * z, axis=0, keepdims=True)


_FBLK = 128
_NFB = L // _FBLK

def _pool_body(zc_ref, zn_ref, s_ref, q_ref, g_ref, b_ref, o_ref):
    i = pl.program_id(1)
    n = float(B * L)
    mu = (s_ref[0] + s_ref[1]) * (1.0 / n)
    var = (q_ref[0] + q_ref[1]) * (1.0 / n) - mu * mu
    sc = g_ref[...] * jax.lax.rsqrt(var + 1e-3)
    zn = _elu((zc_ref[0] - mu) * sc + b_ref[...])
    nx = _elu((zn_ref[0, 0:1, :] - mu) * sc + b_ref[...])
    nx = jnp.where(i == _NFB - 1, NEG, nx)
    z3 = zn.reshape(_FBLK // 2, 2, D)
    ev = z3[:, 0, :]
    od = z3[:, 1, :]
    ev_n = jnp.concatenate([ev[1:, :], nx], axis=0)
    o_ref[0] = jnp.maximum(jnp.maximum(ev, od), ev_n)


def kernel(x, Wq, bq, Wk, bk, Wv, bv, Wo, bo, ln1_g, ln1_b, w1, b1, w2, b2,
           ln2_g, ln2_b, cw, cb, bn_g, bn_b):
    with jax.ensure_compile_time_eval():
        idx = jax.random.randint(jax.random.key(42), (L, U), 0, L)
        At = jnp.zeros((L, L), f32).at[idx, jnp.arange(L)[:, None]].add(1.0)

    Wqkv = jnp.concatenate(
        [Wq.reshape(D, H * DK), Wk.reshape(D, H * DK), Wv.reshape(D, H * DK)],
        axis=1)
    bqkv = jnp.concatenate(
        [bq.reshape(-1), bk.reshape(-1), bv.reshape(-1)])[None, :]

    hspec = pl.BlockSpec((1, H, 256, DK), lambda b, i: (b, 0, i, 0))
    hshape = jax.ShapeDtypeStruct((B, H, L, DK), f32)
    q, k, v = pl.pallas_call(
        _qkv_body,
        grid=(B, L // 256),
        in_specs=[
            pl.BlockSpec((1, 256, D), lambda b, i: (b, i, 0)),
            pl.BlockSpec((D, 3 * D), lambda b, i: (0, 0)),
            pl.BlockSpec((1, 3 * D), lambda b, i: (0, 0)),
        ],
        out_specs=[hspec, hspec, hspec],
        out_shape=[hshape, hshape, hshape],
    )(x, Wqkv, bqkv)

    M = pl.pallas_call(
        _m_body,
        grid=(L // 128,),
        in_specs=[
            pl.BlockSpec((B, H, 128, DK), lambda i: (0, 0, i, 0)),
            pl.BlockSpec((B, H, L, DK), lambda i: (0, 0, 0, 0)),
            pl.BlockSpec((L, 128), lambda i: (0, i)),
        ],
        out_specs=pl.BlockSpec((B * H, 128), lambda i: (0, i)),
        out_shape=jax.ShapeDtypeStruct((B * H, L), f32),
    )(q, k, At)

    top_idx = pl.pallas_call(
        _topk_body,
        in_specs=[pl.BlockSpec((B * H, L), lambda: (0, 0))],
        out_specs=pl.BlockSpec((B * H, U), lambda: (0, 0)),
        out_shape=jax.ShapeDtypeStruct((B * H, U), jnp.int32),
    )(M)

    attn_out = pl.pallas_call(
        _attn_body,
        grid=(B, H),
        in_specs=[
            pl.BlockSpec((1, 1, U), lambda b, h: (b * H + h, 0, 0)),
            pl.BlockSpec((1, 1, L, DK), lambda b, h: (b, h, 0, 0)),
            pl.BlockSpec((1, 1, L, DK), lambda b, h: (b, h, 0, 0)),
            pl.BlockSpec((1, 1, L, DK), lambda b, h: (b, h, 0, 0)),
            pl.BlockSpec((1, DK, D), lambda b, h: (h, 0, 0)),
            pl.BlockSpec((1, D), lambda b, h: (0, 0)),
        ],
        out_specs=pl.BlockSpec((1, L, D), lambda b, h: (b, 0, 0)),
        out_shape=jax.ShapeDtypeStruct((B, L, D), f32),
    )(top_idx.reshape(B * H, 1, U), q, k, v, Wo, bo[None, :])

    y2 = pl.pallas_call(
        _ffn_body,
        grid=(B, L // 256),
        in_specs=[
            pl.BlockSpec((1, 256, D), lambda b, i: (b, i, 0)),
            pl.BlockSpec((1, 256, D), lambda b, i: (b, i, 0)),
            pl.BlockSpec((D, HID), lambda b, i: (0, 0)),
            pl.BlockSpec((1, HID), lambda b, i: (0, 0)),
            pl.BlockSpec((HID, D), lambda b, i: (0, 0)),
            pl.BlockSpec((1, D), lambda b, i: (0, 0)),
            pl.BlockSpec((1, D), lambda b, i: (0, 0)),
            pl.BlockSpec((1, D), lambda b, i: (0, 0)),
            pl.BlockSpec((1, D), lambda b, i: (0, 0)),
            pl.BlockSpec((1, D), lambda b, i: (0, 0)),
        ],
        out_specs=pl.BlockSpec((1, 256, D), lambda b, i: (b, i, 0)),
        out_shape=jax.ShapeDtypeStruct((B, L, D), f32),
    )(x, attn_out, w1, b1[None, :], w2, b2[None, :],
      ln1_g[None, :], ln1_b[None, :], ln2_g[None, :], ln2_b[None, :])

    z, zsum, zsq = pl.pallas_call(
        _conv_body,
        grid=(B,),
        in_specs=[
            pl.BlockSpec((1, L, D), lambda b: (b, 0, 0)),
            pl.BlockSpec((D, D), lambda b: (0, 0)),
            pl.BlockSpec((D, D), lambda b: (0, 0)),
            pl.BlockSpec((D, D), lambda b: (0, 0)),
            pl.BlockSpec((1, D), lambda b: (0, 0)),
        ],
        out_specs=[
            pl.BlockSpec((1, L, D), lambda b: (b, 0, 0)),
            pl.BlockSpec((1, 1, D), lambda b: (b, 0, 0)),
            pl.BlockSpec((1, 1, D), lambda b: (b, 0, 0)),
        ],
        out_shape=[
            jax.ShapeDtypeStruct((B, L, D), f32),
            jax.ShapeDtypeStruct((B, 1, D), f32),
            jax.ShapeDtypeStruct((B, 1, D), f32),
        ],
    )(y2, cw[0], cw[1], cw[2], cb[None, :])

    out = pl.pallas_call(
        _pool_body,
        grid=(B, _NFB),
        in_specs=[
            pl.BlockSpec((1, _FBLK, D), lambda b, i: (b, i, 0)),
            pl.BlockSpec((1, _FBLK, D),
                         lambda b, i: (b, jnp.minimum(i + 1, _NFB - 1), 0)),
            pl.BlockSpec((B, 1, D), lambda b, i: (0, 0, 0)),
            pl.BlockSpec((B, 1, D), lambda b, i: (0, 0, 0)),
            pl.BlockSpec((1, D), lambda b, i: (0, 0)),
            pl.BlockSpec((1, D), lambda b, i: (0, 0)),
        ],
        out_specs=pl.BlockSpec((1, _FBLK // 2, D), lambda b, i: (b, i, 0)),
        out_shape=jax.ShapeDtypeStruct((B, L // 2, D), f32),
    )(z, z, zsum, zsq, bn_g[None, :], bn_b[None, :])

    return out
